# baseline (speedup 1.0000x reference)
import functools

import jax
import jax.numpy as jnp
from jax import lax
from jax.experimental import pallas as pl
from jax.experimental.pallas import tpu as pltpu
from jax.experimental.pallas import tpu_sc as plsc

_B = 16384
_D = 3
_V = 1000000
_L = 16
_GROWS = _V * _D // _L

_info = plsc.get_sparse_core_info()
_NC, _NS = _info.num_cores, _info.num_subcores
_NW = _NC * _NS
_BPW = _B // _NW
_NG = _BPW // _L
_CHUNK = 128
_NCHUNK = 2 * _BPW // _CHUNK
_GPC = _NG // _NCHUNK

_mesh = plsc.VectorSubcoreMesh(core_axis_name="c", subcore_axis_name="s")


@functools.partial(
    pl.kernel,
    mesh=_mesh,
    compiler_params=pltpu.CompilerParams(
        use_tc_tiling_on_sc=False, needs_layout_passes=False
    ),
    out_type=jax.ShapeDtypeStruct((_B * _D,), jnp.float32),
    scratch_types=[
        pltpu.VMEM((_BPW,), jnp.int32),
        [pltpu.VMEM((_CHUNK,), jnp.int32) for _ in range(_NCHUNK)],
        [pltpu.VMEM((_CHUNK, _L), jnp.float32) for _ in range(_NCHUNK)],
        pltpu.VMEM((_BPW * _D,), jnp.float32),
        pltpu.SemaphoreType.DMA,
    ],
)
def _gather_kernel(idx_hbm, table_hbm, out_hbm, idx_v, gidx_vs, win_vs,
                   out_v, sem):
    wid = lax.axis_index("s") * _NC + lax.axis_index("c")
    pltpu.sync_copy(idx_hbm.at[pl.ds(wid * _BPW, _BPW)], idx_v)

    lanes = jnp.arange(_L, dtype=jnp.int32)
    for g in range(_NG):
        v = idx_v[pl.ds(_L * g, _L)]
        t = v * 3
        row0 = t >> 4
        row1 = jnp.minimum(row0 + 1, _GROWS - 1)
        pos = 2 * _L * (g % _GPC) + 2 * lanes
        plsc.store_scatter(gidx_vs[g // _GPC], [pos], row0)
        plsc.store_scatter(gidx_vs[g // _GPC], [pos + 1], row1)

    copies = [
        pltpu.async_copy(table_hbm.at[gidx_vs[j]], win_vs[j], sem)
        for j in range(_NCHUNK)
    ]
    for c in copies:
        c.wait()

    for g in range(_NG):
        v = idx_v[pl.ds(_L * g, _L)]
        t = v * 3
        off = t & 15
        rowbase = 2 * _L * (g % _GPC) + 2 * lanes
        dstbase = _D * (_L * g + lanes)
        for c in range(_D):
            oc = off + c
            vals = plsc.load_gather(
                win_vs[g // _GPC], [rowbase + (oc >> 4), oc & 15]
            )
            plsc.store_scatter(out_v, [dstbase + c], vals)

    pltpu.sync_copy(out_v, out_hbm.at[pl.ds(wid * _BPW * _D, _BPW * _D)])


def kernel(points, delta_translation):
    flat = _gather_kernel(points, delta_translation.reshape(_GROWS, _L))
    return flat.reshape(_B, _D)
